# Initial kernel scaffold; baseline (speedup 1.0000x reference)
import jax
import jax.numpy as jnp
from jax.experimental import pallas as pl


def kernel(x, edge_index, W1, b1, Wmu, bmu, Wsig, bsig):
    raise NotImplementedError("write your pallas kernel here")


# speedup vs baseline: 20.8095x; 20.8095x over previous
import functools

import jax
import jax.numpy as jnp
from jax import lax
from jax.experimental import pallas as pl
from jax.experimental.pallas import tpu as pltpu
from jax.experimental.pallas import tpu_sc as plsc

NC = 2
NS = 16
CHUNK = 128


def _sc_mesh():
    return plsc.VectorSubcoreMesh(
        core_axis_name="c", subcore_axis_name="s", num_cores=NC, num_subcores=NS
    )


def _make_deg_kernel(np_, ep):
    ew = ep // (NC * NS)
    nch = ew // CHUNK
    rows_per_s = np_ // NS

    @functools.partial(
        pl.kernel,
        out_type=jax.ShapeDtypeStruct((NC, np_, 16), jnp.float32),
        mesh=_sc_mesh(),
        scratch_types=[
            pltpu.VMEM((CHUNK, 16), jnp.float32),
            pltpu.VMEM((CHUNK,), jnp.int32),
            pltpu.VMEM_SHARED((np_, 16), jnp.float32),
        ],
    )
    def deg_kernel(dst_hbm, zeros_hbm, out_hbm, ones_v, idx_v, acc):
        c = lax.axis_index("c")
        s = lax.axis_index("s")
        w = c * NS + s

        def fill(j, _):
            ones_v[j] = jnp.full((16,), 1.0, jnp.float32)
            return _

        lax.fori_loop(0, CHUNK, fill, None)
        pltpu.sync_copy(
            zeros_hbm.at[pl.ds(s * rows_per_s, rows_per_s)],
            acc.at[pl.ds(s * rows_per_s, rows_per_s)],
        )
        plsc.subcore_barrier()

        def body(i, _):
            base = w * ew + i * CHUNK
            pltpu.sync_copy(dst_hbm.at[pl.ds(base, CHUNK)], idx_v)
            pltpu.sync_copy(ones_v, acc.at[idx_v], add=True)
            return _

        lax.fori_loop(0, nch, body, None)
        plsc.subcore_barrier()
        pltpu.sync_copy(
            acc.at[pl.ds(s * rows_per_s, rows_per_s)],
            out_hbm.at[c, pl.ds(s * rows_per_s, rows_per_s)],
        )

    return deg_kernel


def _make_prop_kernel(np_, ep):
    ew = ep // (NC * NS)
    nch = ew // CHUNK
    rows_per_s = np_ // NS

    @functools.partial(
        pl.kernel,
        out_type=jax.ShapeDtypeStruct((NC, np_, 64), jnp.float32),
        mesh=_sc_mesh(),
        scratch_types=[
            pltpu.VMEM((CHUNK,), jnp.int32),
            pltpu.VMEM((CHUNK,), jnp.int32),
            pltpu.VMEM((CHUNK, 64), jnp.float32),
            pltpu.VMEM_SHARED((np_, 64), jnp.float32),
        ],
        compiler_params=pltpu.CompilerParams(use_tc_tiling_on_sc=False),
    )
    def prop_kernel(tab_hbm, src_hbm, dst_hbm, zeros_hbm, out_hbm,
                    src_v, dst_v, rows_v, acc):
        c = lax.axis_index("c")
        s = lax.axis_index("s")
        w = c * NS + s

        pltpu.sync_copy(
            zeros_hbm.at[pl.ds(s * rows_per_s, rows_per_s)],
            acc.at[pl.ds(s * rows_per_s, rows_per_s)],
        )
        plsc.subcore_barrier()

        def body(i, _):
            base = w * ew + i * CHUNK
            pltpu.sync_copy(src_hbm.at[pl.ds(base, CHUNK)], src_v)
            pltpu.sync_copy(dst_hbm.at[pl.ds(base, CHUNK)], dst_v)
            pltpu.sync_copy(tab_hbm.at[src_v], rows_v)
            pltpu.sync_copy(rows_v, acc.at[dst_v], add=True)
            return _

        lax.fori_loop(0, nch, body, None)
        plsc.subcore_barrier()
        pltpu.sync_copy(
            acc.at[pl.ds(s * rows_per_s, rows_per_s)],
            out_hbm.at[c, pl.ds(s * rows_per_s, rows_per_s)],
        )

    return prop_kernel


def _tc_pre(x_pad, W1, d0, d1):
    np_ = x_pad.shape[0]

    def body(x_ref, w_ref, d0_ref, d1_ref, hp_ref, dinv_ref):
        deg = d0_ref[...] + d1_ref[...] + 1.0
        dinv = lax.rsqrt(deg)
        dinv_ref[...] = dinv
        h = jnp.dot(x_ref[...], w_ref[...], preferred_element_type=jnp.float32)
        hp_ref[...] = h * dinv[:, :1]

    return pl.pallas_call(
        body,
        out_shape=(
            jax.ShapeDtypeStruct((np_, 64), jnp.float32),
            jax.ShapeDtypeStruct((np_, 16), jnp.float32),
        ),
    )(x_pad, W1, d0, d1)


def _tc_mid(u0, u1, hp, dinv, b1):
    np_ = hp.shape[0]

    def body(u0_ref, u1_ref, hp_ref, dinv_ref, b_ref, out_ref):
        t = dinv_ref[...][:, :1]
        pre = (u0_ref[...] + u1_ref[...] + hp_ref[...]) * t + b_ref[...]
        out_ref[...] = jnp.maximum(pre, 0.0) * t

    return pl.pallas_call(
        body,
        out_shape=jax.ShapeDtypeStruct((np_, 64), jnp.float32),
    )(u0, u1, hp, dinv, b1)


def _tc_heads(u0, u1, hp, dinv, Wmu, bmu, Wsig, bsig):
    np_ = hp.shape[0]

    def body(u0_ref, u1_ref, hp_ref, dinv_ref, wm_ref, bm_ref, ws_ref,
             bs_ref, mu_ref, sig_ref):
        g = (u0_ref[...] + u1_ref[...] + hp_ref[...]) * dinv_ref[...][:, :1]
        mu_ref[...] = jnp.dot(g, wm_ref[...],
                              preferred_element_type=jnp.float32) + bm_ref[...]
        sig_ref[...] = jnp.dot(g, ws_ref[...],
                               preferred_element_type=jnp.float32) + bs_ref[...]

    return pl.pallas_call(
        body,
        out_shape=(
            jax.ShapeDtypeStruct((np_, 32), jnp.float32),
            jax.ShapeDtypeStruct((np_, 32), jnp.float32),
        ),
    )(u0, u1, hp, dinv, Wmu, bmu, Wsig, bsig)


@jax.jit
def kernel(x, edge_index, W1, b1, Wmu, bmu, Wsig, bsig):
    n = x.shape[0]
    e = edge_index.shape[1]
    np_ = ((n + 16 + 127) // 128) * 128

    quant = NC * NS * CHUNK
    ep = ((e + quant - 1) // quant) * quant
    pad_n = ep - e
    pad_idx = n + (jnp.arange(pad_n, dtype=edge_index.dtype) % 16)
    src = jnp.concatenate([edge_index[0], pad_idx])
    dst = jnp.concatenate([edge_index[1], pad_idx])

    zeros16 = jnp.zeros((np_, 16), jnp.float32)
    zeros64 = jnp.zeros((np_, 64), jnp.float32)
    x_pad = jnp.concatenate([x, jnp.zeros((np_ - n, x.shape[1]), x.dtype)])

    dpart = _make_deg_kernel(np_, ep)(dst, zeros16)
    prop = _make_prop_kernel(np_, ep)

    hp, dinv = _tc_pre(x_pad, W1, dpart[0], dpart[1])
    u = prop(hp, src, dst, zeros64)
    h2p = _tc_mid(u[0], u[1], hp, dinv, b1.reshape(1, 64))
    u2 = prop(h2p, src, dst, zeros64)
    mu, sig = _tc_heads(u2[0], u2[1], h2p, dinv, Wmu, bmu.reshape(1, 32),
                        Wsig, bsig.reshape(1, 32))
    return (mu[:n], sig[:n])


# speedup vs baseline: 39.6904x; 1.9073x over previous
import functools

import jax
import jax.numpy as jnp
from jax import lax
from jax.experimental import pallas as pl
from jax.experimental.pallas import tpu as pltpu
from jax.experimental.pallas import tpu_sc as plsc

NC = 2
NS = 16
CHUNK = 128


def _sc_mesh():
    return plsc.VectorSubcoreMesh(
        core_axis_name="c", subcore_axis_name="s", num_cores=NC, num_subcores=NS
    )


def _zero_fill(buf, nrows, width):
    per_row = width // 16

    def body(t, carry):
        buf[t // per_row, pl.ds((t % per_row) * 16, 16)] = jnp.zeros(
            (16,), jnp.float32)
        return carry

    lax.fori_loop(0, nrows * per_row, body, None)


def _zero_acc(acc, zbuf, row0, nrows):
    nfull, rem = nrows // 128, nrows % 128
    for i in range(nfull):
        pltpu.sync_copy(zbuf, acc.at[pl.ds(row0 + i * 128, 128)])
    if rem:
        pltpu.sync_copy(zbuf.at[pl.ds(0, rem)],
                        acc.at[pl.ds(row0 + nfull * 128, rem)])


def _make_deg_kernel(np_, ep):
    ew = ep // (NC * NS)
    nch = ew // CHUNK
    rows_per_s = np_ // NS

    @functools.partial(
        pl.kernel,
        out_type=jax.ShapeDtypeStruct((NC, np_, 16), jnp.float32),
        mesh=_sc_mesh(),
        scratch_types=[
            pltpu.VMEM((CHUNK, 16), jnp.float32),
            pltpu.VMEM((CHUNK, 16), jnp.float32),
            pltpu.VMEM((nch, CHUNK), jnp.int32),
            pltpu.VMEM_SHARED((np_, 16), jnp.float32),
            pltpu.SemaphoreType.DMA,
        ],
        compiler_params=pltpu.CompilerParams(use_tc_tiling_on_sc=False),
    )
    def deg_kernel(dst_hbm, out_hbm, ones_v, zbuf, idx_v, acc, sem):
        c = lax.axis_index("c")
        s = lax.axis_index("s")
        w = c * NS + s

        def fill(j, carry):
            ones_v[j] = jnp.full((16,), 1.0, jnp.float32)
            return carry

        lax.fori_loop(0, CHUNK, fill, None)
        _zero_fill(zbuf, CHUNK, 16)
        pltpu.sync_copy(dst_hbm.at[pl.ds(w * nch, nch)], idx_v)
        _zero_acc(acc, zbuf, s * rows_per_s, rows_per_s)
        plsc.subcore_barrier()

        def outer(g, carry):
            for j in range(8):
                cc = g * 8 + j

                @pl.when(cc < nch)
                def _():
                    pltpu.async_copy(ones_v, acc.at[idx_v.at[cc]], sem,
                                     add=True)

            for j in range(8):
                cc = g * 8 + j

                @pl.when(cc < nch)
                def _():
                    pltpu.make_async_copy(
                        ones_v, acc.at[idx_v.at[0]], sem).wait()

            return carry

        lax.fori_loop(0, (nch + 7) // 8, outer, None)
        plsc.subcore_barrier()
        pltpu.sync_copy(
            acc.at[pl.ds(s * rows_per_s, rows_per_s)],
            out_hbm.at[c, pl.ds(s * rows_per_s, rows_per_s)],
        )

    return deg_kernel


def _make_prop_kernel(np_, ep, n_tab):
    ew = ep // (NC * NS)
    nch = ew // CHUNK
    rows_per_s = np_ // NS
    nslots = ((nch + 3) // 2) * 2

    @functools.partial(
        pl.kernel,
        out_type=jax.ShapeDtypeStruct((NC, np_, 64), jnp.float32),
        mesh=_sc_mesh(),
        scratch_types=[
            pltpu.VMEM((CHUNK, 64), jnp.float32),
            pltpu.VMEM((CHUNK, 64), jnp.float32),
            pltpu.VMEM((nch, CHUNK), jnp.int32),
            pltpu.VMEM((nch, CHUNK), jnp.int32),
            pltpu.VMEM_SHARED((np_, 64), jnp.float32),
            pltpu.SemaphoreType.DMA,
            pltpu.SemaphoreType.DMA,
            pltpu.SemaphoreType.DMA,
            pltpu.SemaphoreType.DMA,
        ],
        compiler_params=pltpu.CompilerParams(use_tc_tiling_on_sc=False),
    )
    def prop_kernel(tab_hbm, src_hbm, dst_hbm, out_hbm,
                    gbuf0, gbuf1, src_v, dst_v, acc, sg0, sg1, ss0, ss1):
        c = lax.axis_index("c")
        s = lax.axis_index("s")
        w = c * NS + s
        gbuf = (gbuf0, gbuf1)
        sg = (sg0, sg1)
        ss = (ss0, ss1)

        _zero_fill(gbuf0, CHUNK, 64)
        pltpu.sync_copy(src_hbm.at[pl.ds(w * nch, nch)], src_v)
        pltpu.sync_copy(dst_hbm.at[pl.ds(w * nch, nch)], dst_v)
        _zero_acc(acc, gbuf0, s * rows_per_s, rows_per_s)
        plsc.subcore_barrier()

        def outer(g, carry):
            for b in range(2):
                t = g * 2 + b

                @pl.when((t >= 2) & (t <= nch + 1))
                def _():
                    pltpu.make_async_copy(
                        gbuf[b], acc.at[dst_v.at[0]], ss[b]).wait()

                @pl.when(t < nch)
                def _():
                    pltpu.async_copy(tab_hbm.at[src_v.at[t]], gbuf[b], sg[b])

                @pl.when((t >= 1) & (t <= nch))
                def _():
                    pltpu.make_async_copy(
                        tab_hbm.at[src_v.at[0]], gbuf[1 - b],
                        sg[1 - b]).wait()
                    pltpu.async_copy(gbuf[1 - b], acc.at[dst_v.at[t - 1]],
                                     ss[1 - b], add=True)

            return carry

        lax.fori_loop(0, nslots // 2, outer, None)
        plsc.subcore_barrier()
        pltpu.sync_copy(
            acc.at[pl.ds(s * rows_per_s, rows_per_s)],
            out_hbm.at[c, pl.ds(s * rows_per_s, rows_per_s)],
        )

    return prop_kernel


def _tc_pre(x, W1, d0, d1):
    n = x.shape[0]

    def body(x_ref, w_ref, d0_ref, d1_ref, hp_ref, dinv_ref):
        deg = d0_ref[...] + d1_ref[...] + 1.0
        dinv = lax.rsqrt(deg)
        dinv_ref[...] = dinv
        h = jnp.dot(x_ref[...], w_ref[...], preferred_element_type=jnp.float32)
        hp_ref[...] = h * dinv[:, :1]

    return pl.pallas_call(
        body,
        out_shape=(
            jax.ShapeDtypeStruct((n, 64), jnp.float32),
            jax.ShapeDtypeStruct((n, 16), jnp.float32),
        ),
    )(x, W1, d0, d1)


def _tc_mid(u0, u1, hp, dinv, b1):
    n = hp.shape[0]

    def body(u0_ref, u1_ref, hp_ref, dinv_ref, b_ref, out_ref):
        t = dinv_ref[...][:, :1]
        pre = (u0_ref[...] + u1_ref[...] + hp_ref[...]) * t + b_ref[...]
        out_ref[...] = jnp.maximum(pre, 0.0) * t

    return pl.pallas_call(
        body,
        out_shape=jax.ShapeDtypeStruct((n, 64), jnp.float32),
    )(u0, u1, hp, dinv, b1)


def _tc_heads(u0, u1, hp, dinv, Wmu, bmu, Wsig, bsig):
    n = hp.shape[0]

    def body(u0_ref, u1_ref, hp_ref, dinv_ref, wm_ref, bm_ref, ws_ref,
             bs_ref, mu_ref, sig_ref):
        g = (u0_ref[...] + u1_ref[...] + hp_ref[...]) * dinv_ref[...][:, :1]
        mu_ref[...] = jnp.dot(g, wm_ref[...],
                              preferred_element_type=jnp.float32) + bm_ref[...]
        sig_ref[...] = jnp.dot(g, ws_ref[...],
                               preferred_element_type=jnp.float32) + bs_ref[...]

    return pl.pallas_call(
        body,
        out_shape=(
            jax.ShapeDtypeStruct((n, 32), jnp.float32),
            jax.ShapeDtypeStruct((n, 32), jnp.float32),
        ),
    )(u0, u1, hp, dinv, Wmu, bmu, Wsig, bsig)


@jax.jit
def kernel(x, edge_index, W1, b1, Wmu, bmu, Wsig, bsig):
    n = x.shape[0]
    e = edge_index.shape[1]
    np_ = ((n + 16 + 127) // 128) * 128

    quant = NC * NS * CHUNK * 8
    ep = ((e + quant - 1) // quant) * quant
    pad_n = ep - e
    ar = jnp.arange(pad_n, dtype=edge_index.dtype)
    src = jnp.concatenate([edge_index[0], ar % 16]).reshape(ep // CHUNK, CHUNK)
    dst = jnp.concatenate([edge_index[1], n + ar % (np_ - n)]).reshape(
        ep // CHUNK, CHUNK)

    x_pad = jnp.concatenate([x, jnp.zeros((np_ - n, x.shape[1]), x.dtype)])

    dpart = _make_deg_kernel(np_, ep)(dst)
    prop = _make_prop_kernel(np_, ep, n)

    hp, dinv = _tc_pre(x_pad, W1, dpart[0], dpart[1])
    u = prop(hp, src, dst)
    h2p = _tc_mid(u[0], u[1], hp, dinv, b1.reshape(1, 64))
    u2 = prop(h2p, src, dst)
    mu, sig = _tc_heads(u2[0], u2[1], h2p, dinv, Wmu,
                        bmu.reshape(1, 32), Wsig, bsig.reshape(1, 32))
    return (mu[:n], sig[:n])


# speedup vs baseline: 41.2862x; 1.0402x over previous
import functools

import jax
import jax.numpy as jnp
from jax import lax
from jax.experimental import pallas as pl
from jax.experimental.pallas import tpu as pltpu
from jax.experimental.pallas import tpu_sc as plsc

NC = 2
NS = 16
CHUNK = 128


def _sc_mesh():
    return plsc.VectorSubcoreMesh(
        core_axis_name="c", subcore_axis_name="s", num_cores=NC, num_subcores=NS
    )


def _zero_fill(buf, nrows, width):
    per_row = width // 16

    def body(t, carry):
        buf[t // per_row, pl.ds((t % per_row) * 16, 16)] = jnp.zeros(
            (16,), jnp.float32)
        return carry

    lax.fori_loop(0, nrows * per_row, body, None)


def _zero_acc(acc, zbuf, row0, nrows):
    nfull, rem = nrows // 128, nrows % 128
    for i in range(nfull):
        pltpu.sync_copy(zbuf, acc.at[pl.ds(row0 + i * 128, 128)])
    if rem:
        pltpu.sync_copy(zbuf.at[pl.ds(0, rem)],
                        acc.at[pl.ds(row0 + nfull * 128, rem)])


def _make_deg_kernel(np_, ep):
    ew = ep // (NC * NS)
    nch = ew // CHUNK
    rows_per_s = np_ // NS

    @functools.partial(
        pl.kernel,
        out_type=jax.ShapeDtypeStruct((NC, np_, 16), jnp.float32),
        mesh=_sc_mesh(),
        scratch_types=[
            pltpu.VMEM((CHUNK, 16), jnp.float32),
            pltpu.VMEM((CHUNK, 16), jnp.float32),
            pltpu.VMEM((nch, CHUNK), jnp.int32),
            pltpu.VMEM_SHARED((np_, 16), jnp.float32),
            pltpu.SemaphoreType.DMA,
        ],
        compiler_params=pltpu.CompilerParams(use_tc_tiling_on_sc=False),
    )
    def deg_kernel(dst_hbm, out_hbm, ones_v, zbuf, idx_v, acc, sem):
        c = lax.axis_index("c")
        s = lax.axis_index("s")
        w = c * NS + s

        def fill(j, carry):
            ones_v[j] = jnp.full((16,), 1.0, jnp.float32)
            return carry

        lax.fori_loop(0, CHUNK, fill, None)
        _zero_fill(zbuf, CHUNK, 16)
        pltpu.sync_copy(dst_hbm.at[pl.ds(w * nch, nch)], idx_v)
        _zero_acc(acc, zbuf, s * rows_per_s, rows_per_s)
        plsc.subcore_barrier()

        def outer(g, carry):
            for j in range(8):
                cc = g * 8 + j

                @pl.when(cc < nch)
                def _():
                    pltpu.async_copy(ones_v, acc.at[idx_v.at[cc]], sem,
                                     add=True)

            for j in range(8):
                cc = g * 8 + j

                @pl.when(cc < nch)
                def _():
                    pltpu.make_async_copy(
                        ones_v, acc.at[idx_v.at[0]], sem).wait()

            return carry

        lax.fori_loop(0, (nch + 7) // 8, outer, None)
        plsc.subcore_barrier()
        pltpu.sync_copy(
            acc.at[pl.ds(s * rows_per_s, rows_per_s)],
            out_hbm.at[c, pl.ds(s * rows_per_s, rows_per_s)],
        )

    return deg_kernel


def _make_prop_kernel(np_, ep, n_tab):
    ew = ep // (NC * NS)
    nch = ew // CHUNK
    rows_per_s = np_ // NS
    nbuf = 4
    nslots = ((nch + nbuf + nbuf - 1) // nbuf) * nbuf

    @functools.partial(
        pl.kernel,
        out_type=jax.ShapeDtypeStruct((NC, np_, 64), jnp.float32),
        mesh=_sc_mesh(),
        scratch_types=[
            pltpu.VMEM((nbuf, CHUNK, 64), jnp.float32),
            pltpu.VMEM((nch, CHUNK), jnp.int32),
            pltpu.VMEM((nch, CHUNK), jnp.int32),
            pltpu.VMEM_SHARED((np_, 64), jnp.float32),
        ] + [pltpu.SemaphoreType.DMA] * (2 * nbuf),
        compiler_params=pltpu.CompilerParams(use_tc_tiling_on_sc=False),
    )
    def prop_kernel(tab_hbm, src_hbm, dst_hbm, out_hbm,
                    gbufs, src_v, dst_v, acc, *sems):
        c = lax.axis_index("c")
        s = lax.axis_index("s")
        w = c * NS + s
        sg = sems[:nbuf]
        ss = sems[nbuf:]

        _zero_fill(gbufs.at[0], CHUNK, 64)
        pltpu.sync_copy(src_hbm.at[pl.ds(w * nch, nch)], src_v)
        pltpu.sync_copy(dst_hbm.at[pl.ds(w * nch, nch)], dst_v)
        _zero_acc(acc, gbufs.at[0], s * rows_per_s, rows_per_s)
        plsc.subcore_barrier()

        def outer(g, carry):
            for j in range(nbuf):
                t = g * nbuf + j
                b = j
                b2 = (j + nbuf - 2) % nbuf

                @pl.when((t >= nbuf) & (t <= nch + nbuf - 1))
                def _():
                    pltpu.make_async_copy(
                        gbufs.at[b], acc.at[dst_v.at[0]], ss[b]).wait()

                @pl.when(t < nch)
                def _():
                    pltpu.async_copy(tab_hbm.at[src_v.at[t]], gbufs.at[b],
                                     sg[b])

                @pl.when((t >= 2) & (t <= nch + 1))
                def _():
                    pltpu.make_async_copy(
                        tab_hbm.at[src_v.at[0]], gbufs.at[b2],
                        sg[b2]).wait()
                    pltpu.async_copy(gbufs.at[b2], acc.at[dst_v.at[t - 2]],
                                     ss[b2], add=True)

            return carry

        lax.fori_loop(0, nslots // nbuf, outer, None)
        plsc.subcore_barrier()
        pltpu.sync_copy(
            acc.at[pl.ds(s * rows_per_s, rows_per_s)],
            out_hbm.at[c, pl.ds(s * rows_per_s, rows_per_s)],
        )

    return prop_kernel


def _tc_pre(x, W1, d0, d1):
    n = x.shape[0]

    def body(x_ref, w_ref, d0_ref, d1_ref, hp_ref, dinv_ref):
        deg = d0_ref[...] + d1_ref[...] + 1.0
        dinv = lax.rsqrt(deg)
        dinv_ref[...] = dinv
        h = jnp.dot(x_ref[...], w_ref[...], preferred_element_type=jnp.float32)
        hp_ref[...] = h * dinv[:, :1]

    return pl.pallas_call(
        body,
        out_shape=(
            jax.ShapeDtypeStruct((n, 64), jnp.float32),
            jax.ShapeDtypeStruct((n, 16), jnp.float32),
        ),
    )(x, W1, d0, d1)


def _tc_mid(u0, u1, hp, dinv, b1):
    n = hp.shape[0]

    def body(u0_ref, u1_ref, hp_ref, dinv_ref, b_ref, out_ref):
        t = dinv_ref[...][:, :1]
        pre = (u0_ref[...] + u1_ref[...] + hp_ref[...]) * t + b_ref[...]
        out_ref[...] = jnp.maximum(pre, 0.0) * t

    return pl.pallas_call(
        body,
        out_shape=jax.ShapeDtypeStruct((n, 64), jnp.float32),
    )(u0, u1, hp, dinv, b1)


def _tc_heads(u0, u1, hp, dinv, Wmu, bmu, Wsig, bsig):
    n = hp.shape[0]

    def body(u0_ref, u1_ref, hp_ref, dinv_ref, wm_ref, bm_ref, ws_ref,
             bs_ref, mu_ref, sig_ref):
        g = (u0_ref[...] + u1_ref[...] + hp_ref[...]) * dinv_ref[...][:, :1]
        mu_ref[...] = jnp.dot(g, wm_ref[...],
                              preferred_element_type=jnp.float32) + bm_ref[...]
        sig_ref[...] = jnp.dot(g, ws_ref[...],
                               preferred_element_type=jnp.float32) + bs_ref[...]

    return pl.pallas_call(
        body,
        out_shape=(
            jax.ShapeDtypeStruct((n, 32), jnp.float32),
            jax.ShapeDtypeStruct((n, 32), jnp.float32),
        ),
    )(u0, u1, hp, dinv, Wmu, bmu, Wsig, bsig)


@jax.jit
def kernel(x, edge_index, W1, b1, Wmu, bmu, Wsig, bsig):
    n = x.shape[0]
    e = edge_index.shape[1]
    np_ = ((n + 16 + 127) // 128) * 128

    quant = NC * NS * CHUNK * 8
    ep = ((e + quant - 1) // quant) * quant
    pad_n = ep - e
    ar = jnp.arange(pad_n, dtype=edge_index.dtype)
    src = jnp.concatenate([edge_index[0], ar % 16]).reshape(ep // CHUNK, CHUNK)
    dst = jnp.concatenate([edge_index[1], n + ar % (np_ - n)]).reshape(
        ep // CHUNK, CHUNK)

    x_pad = jnp.concatenate([x, jnp.zeros((np_ - n, x.shape[1]), x.dtype)])

    dpart = _make_deg_kernel(np_, ep)(dst)
    prop = _make_prop_kernel(np_, ep, n)

    hp, dinv = _tc_pre(x_pad, W1, dpart[0], dpart[1])
    u = prop(hp, src, dst)
    h2p = _tc_mid(u[0], u[1], hp, dinv, b1.reshape(1, 64))
    u2 = prop(h2p, src, dst)
    mu, sig = _tc_heads(u2[0], u2[1], h2p, dinv, Wmu,
                        bmu.reshape(1, 32), Wsig, bsig.reshape(1, 32))
    return (mu[:n], sig[:n])


# speedup vs baseline: 49.9182x; 1.2091x over previous
import functools

import jax
import jax.numpy as jnp
from jax import lax
from jax.experimental import pallas as pl
from jax.experimental.pallas import tpu as pltpu
from jax.experimental.pallas import tpu_sc as plsc

NC = 2
NS = 16
CHUNK = 128


def _sc_mesh():
    return plsc.VectorSubcoreMesh(
        core_axis_name="c", subcore_axis_name="s", num_cores=NC, num_subcores=NS
    )


def _zero_fill(buf, nrows, width):
    per_row = width // 16

    def body(t, carry):
        buf[t // per_row, pl.ds((t % per_row) * 16, 16)] = jnp.zeros(
            (16,), jnp.float32)
        return carry

    lax.fori_loop(0, nrows * per_row, body, None)


def _zero_acc(acc, zbuf, row0, nrows):
    nfull, rem = nrows // 128, nrows % 128
    for i in range(nfull):
        pltpu.sync_copy(zbuf, acc.at[pl.ds(row0 + i * 128, 128)])
    if rem:
        pltpu.sync_copy(zbuf.at[pl.ds(0, rem)],
                        acc.at[pl.ds(row0 + nfull * 128, rem)])


def _make_deg_kernel(np_, ep):
    ew = ep // (NC * NS)
    nch = ew // CHUNK
    rows_per_s = np_ // NS

    @functools.partial(
        pl.kernel,
        out_type=jax.ShapeDtypeStruct((NC, np_, 16), jnp.float32),
        mesh=_sc_mesh(),
        scratch_types=[
            pltpu.VMEM((CHUNK, 16), jnp.float32),
            pltpu.VMEM((CHUNK, 16), jnp.float32),
            pltpu.VMEM((nch, CHUNK), jnp.int32),
            pltpu.VMEM_SHARED((np_, 16), jnp.float32),
            pltpu.SemaphoreType.DMA,
        ],
        compiler_params=pltpu.CompilerParams(use_tc_tiling_on_sc=False),
    )
    def deg_kernel(dst_hbm, out_hbm, ones_v, zbuf, idx_v, acc, sem):
        c = lax.axis_index("c")
        s = lax.axis_index("s")
        w = c * NS + s

        def fill(j, carry):
            ones_v[j] = jnp.full((16,), 1.0, jnp.float32)
            return carry

        lax.fori_loop(0, CHUNK, fill, None)
        _zero_fill(zbuf, CHUNK, 16)
        pltpu.sync_copy(dst_hbm.at[pl.ds(w * nch, nch)], idx_v)
        _zero_acc(acc, zbuf, s * rows_per_s, rows_per_s)
        plsc.subcore_barrier()

        def outer(g, carry):
            for j in range(8):
                cc = g * 8 + j

                @pl.when(cc < nch)
                def _():
                    pltpu.async_copy(ones_v, acc.at[idx_v.at[cc]], sem,
                                     add=True)

            for j in range(8):
                cc = g * 8 + j

                @pl.when(cc < nch)
                def _():
                    pltpu.make_async_copy(
                        ones_v, acc.at[idx_v.at[0]], sem).wait()

            return carry

        lax.fori_loop(0, (nch + 7) // 8, outer, None)
        plsc.subcore_barrier()
        pltpu.sync_copy(
            acc.at[pl.ds(s * rows_per_s, rows_per_s)],
            out_hbm.at[c, pl.ds(s * rows_per_s, rows_per_s)],
        )

    return deg_kernel


def _make_prop_kernel(np_, ep, n_tab):
    ew = ep // (NC * NS)
    nch = ew // CHUNK
    rows_per_s = np_ // NS
    nbuf = 4
    nslots = ((nch + nbuf + nbuf - 1) // nbuf) * nbuf

    @functools.partial(
        pl.kernel,
        out_type=jax.ShapeDtypeStruct((NC, np_, 64), jnp.float32),
        mesh=_sc_mesh(),
        scratch_types=[
            pltpu.VMEM((nbuf, CHUNK, 64), jnp.float32),
            pltpu.VMEM((nch, CHUNK), jnp.int32),
            pltpu.VMEM((nch, CHUNK), jnp.int32),
            pltpu.VMEM_SHARED((np_, 64), jnp.float32),
        ] + [pltpu.SemaphoreType.DMA] * (2 * nbuf),
        compiler_params=pltpu.CompilerParams(use_tc_tiling_on_sc=False),
    )
    def prop_kernel(tab_hbm, src_hbm, dst_hbm, out_hbm,
                    gbufs, src_v, dst_v, acc, *sems):
        c = lax.axis_index("c")
        s = lax.axis_index("s")
        w = c * NS + s
        sg = sems[:nbuf]
        ss = sems[nbuf:]

        _zero_fill(gbufs.at[0], CHUNK, 64)
        pltpu.sync_copy(src_hbm.at[pl.ds(w * nch, nch)], src_v)
        pltpu.sync_copy(dst_hbm.at[pl.ds(w * nch, nch)], dst_v)
        _zero_acc(acc, gbufs.at[0], s * rows_per_s, rows_per_s)
        plsc.subcore_barrier()

        def outer(g, carry):
            for j in range(nbuf):
                t = g * nbuf + j
                b = j
                b2 = (j + nbuf - 2) % nbuf

                @pl.when((t >= nbuf) & (t <= nch + nbuf - 1))
                def _():
                    pltpu.make_async_copy(
                        gbufs.at[b], acc.at[dst_v.at[0]], ss[b]).wait()

                @pl.when(t < nch)
                def _():
                    pltpu.async_copy(tab_hbm.at[src_v.at[t]], gbufs.at[b],
                                     sg[b])

                @pl.when((t >= 2) & (t <= nch + 1))
                def _():
                    pltpu.make_async_copy(
                        tab_hbm.at[src_v.at[0]], gbufs.at[b2],
                        sg[b2]).wait()
                    pltpu.async_copy(gbufs.at[b2], acc.at[dst_v.at[t - 2]],
                                     ss[b2], add=True)

            return carry

        lax.fori_loop(0, nslots // nbuf, outer, None)
        plsc.subcore_barrier()
        pltpu.sync_copy(
            acc.at[pl.ds(s * rows_per_s, rows_per_s)],
            out_hbm.at[c, pl.ds(s * rows_per_s, rows_per_s)],
        )

    return prop_kernel


def _tc_pre(x, W1, dpart):
    n = x.shape[0]

    def body(x_ref, w_ref, dp_ref, hp_ref, dinv_ref):
        deg = dp_ref[0, :n, :] + dp_ref[1, :n, :] + 1.0
        dinv = lax.rsqrt(deg)
        dinv_ref[...] = dinv
        h = jnp.dot(x_ref[...], w_ref[...], preferred_element_type=jnp.float32)
        hp_ref[...] = h * dinv[:, :1]

    return pl.pallas_call(
        body,
        out_shape=(
            jax.ShapeDtypeStruct((n, 64), jnp.float32),
            jax.ShapeDtypeStruct((n, 16), jnp.float32),
        ),
    )(x, W1, dpart)


def _tc_mid(u, hp, dinv, b1):
    n = hp.shape[0]

    def body(u_ref, hp_ref, dinv_ref, b_ref, out_ref):
        t = dinv_ref[...][:, :1]
        usum = u_ref[0, :n, :] + u_ref[1, :n, :]
        pre = (usum + hp_ref[...]) * t + b_ref[...]
        out_ref[...] = jnp.maximum(pre, 0.0) * t

    return pl.pallas_call(
        body,
        out_shape=jax.ShapeDtypeStruct((n, 64), jnp.float32),
    )(u, hp, dinv, b1)


def _tc_heads(u, hp, dinv, Wmu, bmu, Wsig, bsig):
    n = hp.shape[0]

    def body(u_ref, hp_ref, dinv_ref, wm_ref, bm_ref, ws_ref,
             bs_ref, mu_ref, sig_ref):
        usum = u_ref[0, :n, :] + u_ref[1, :n, :]
        g = (usum + hp_ref[...]) * dinv_ref[...][:, :1]
        mu_ref[...] = jnp.dot(g, wm_ref[...],
                              preferred_element_type=jnp.float32) + bm_ref[...]
        sig_ref[...] = jnp.dot(g, ws_ref[...],
                               preferred_element_type=jnp.float32) + bs_ref[...]

    return pl.pallas_call(
        body,
        out_shape=(
            jax.ShapeDtypeStruct((n, 32), jnp.float32),
            jax.ShapeDtypeStruct((n, 32), jnp.float32),
        ),
    )(u, hp, dinv, Wmu, bmu, Wsig, bsig)


@jax.jit
def kernel(x, edge_index, W1, b1, Wmu, bmu, Wsig, bsig):
    n = x.shape[0]
    e = edge_index.shape[1]
    np_ = ((n + 16 + 127) // 128) * 128

    quant = NC * NS * CHUNK * 8
    ep = ((e + quant - 1) // quant) * quant
    pad_n = ep - e
    ar = jnp.arange(pad_n, dtype=edge_index.dtype)
    src = jnp.concatenate([edge_index[0], ar % n]).reshape(ep // CHUNK, CHUNK)
    dst = jnp.concatenate([edge_index[1], n + ar % (np_ - n)]).reshape(
        ep // CHUNK, CHUNK)

    dpart = _make_deg_kernel(np_, ep)(dst)
    prop = _make_prop_kernel(np_, ep, n)

    hp, dinv = _tc_pre(x, W1, dpart)
    u = prop(hp, src, dst)
    h2p = _tc_mid(u, hp, dinv, b1.reshape(1, 64))
    u2 = prop(h2p, src, dst)
    mu, sig = _tc_heads(u2, h2p, dinv, Wmu,
                        bmu.reshape(1, 32), Wsig, bsig.reshape(1, 32))
    return (mu, sig)
